# baseline (speedup 1.0000x reference)
import functools

import jax
import jax.numpy as jnp
from jax import lax
from jax.experimental import pallas as pl
from jax.experimental.pallas import tpu as pltpu
from jax.experimental.pallas import tpu_sc as plsc

_B = 256
_N = 625
_NPAD = 640
_SIZE = 25
_STRIDE = 8.0
_ORI = -96.0
_NW = 32
_RPW = _B // _NW
_NCHUNK = _NPAD // 16


def _polar_pick_sc(score, locr):
    mesh = plsc.VectorSubcoreMesh(core_axis_name="c", subcore_axis_name="s")

    @functools.partial(
        pl.kernel,
        mesh=mesh,
        out_type=jax.ShapeDtypeStruct((_B * 2,), jnp.float32),
        compiler_params=pltpu.CompilerParams(needs_layout_passes=False),
        scratch_types=[
            pltpu.VMEM((_RPW * _NPAD,), jnp.float32),
            pltpu.VMEM((_RPW * 4 * _N,), jnp.float32),
            pltpu.VMEM((16,), jnp.float32),
            pltpu.SemaphoreType.DMA,
        ],
    )
    def k(score_hbm, loc_hbm, out_hbm, score_v, loc_v, out_v, sem):
        c = lax.axis_index("c")
        s = lax.axis_index("s")
        w = s * 2 + c
        base = w * _RPW
        loc_cp = pltpu.async_copy(
            loc_hbm.at[pl.ds(base * 4 * _N, _RPW * 4 * _N)], loc_v, sem)
        pltpu.sync_copy(
            score_hbm.at[pl.ds(base * _NPAD, _RPW * _NPAD)], score_v)

        lane = lax.iota(jnp.int32, 16)
        row_l = lane >> 1

        def _allreduce(v, binop):
            for step in (1, 2, 4, 8):
                shuf = v.at[lane ^ step].get(mode="promise_in_bounds")
                v = binop(v, shuf)
            return v

        idx_pair = jnp.zeros((16,), jnp.int32)
        for r in range(_RPW):
            vmax = score_v[pl.ds(r * _NPAD, 16)]
            vidx = lane
            for chunk in range(1, _NCHUNK):
                v = score_v[pl.ds(r * _NPAD + chunk * 16, 16)]
                gt = v > vmax
                vmax = jnp.where(gt, v, vmax)
                vidx = jnp.where(gt, lane + chunk * 16, vidx)
            m = _allreduce(vmax, jnp.maximum)
            cand = jnp.where(vmax == m, vidx, jnp.int32(2**30))
            idx_vec = _allreduce(cand, jnp.minimum)
            idx_pair = jnp.where(row_l == r, idx_vec, idx_pair)
        kbit = lane & 1
        loc_cp.wait()
        flat1 = (row_l * 4 + kbit) * _N + idx_pair
        g1 = plsc.load_gather(loc_v, [flat1])
        g2 = plsc.load_gather(loc_v, [flat1 + 2 * _N])
        sel = jnp.where(kbit == 0, idx_pair % _SIZE, idx_pair // _SIZE)
        p = sel.astype(jnp.float32) * jnp.float32(_STRIDE) + jnp.float32(_ORI)
        out_v[...] = p + (g2 - g1) * jnp.float32(0.5)
        pltpu.sync_copy(out_v, out_hbm.at[pl.ds(base * 2, 16)])

    return k(score, locr)


def kernel(cls, loc):
    score = cls.reshape(_B, 2, _N)[:, 1, :]
    score = jnp.pad(score, ((0, 0), (0, _NPAD - _N)),
                    constant_values=float("-inf"))
    out = _polar_pick_sc(score.reshape(-1), loc.reshape(-1))
    return out.reshape(_B, 2)


# speedup vs baseline: 1.0018x; 1.0018x over previous
import functools

import jax
import jax.numpy as jnp
from jax import lax
from jax.experimental import pallas as pl
from jax.experimental.pallas import tpu as pltpu
from jax.experimental.pallas import tpu_sc as plsc

_B = 256
_N = 625
_NPAD = 640
_SIZE = 25
_STRIDE = 8.0
_ORI = -96.0
_NW = 32
_RPW = _B // _NW
_NCHUNK = _NPAD // 16


def _polar_pick_sc(score, locr):
    mesh = plsc.VectorSubcoreMesh(core_axis_name="c", subcore_axis_name="s")

    @functools.partial(
        pl.kernel,
        mesh=mesh,
        out_type=jax.ShapeDtypeStruct((_B * 2,), jnp.float32),
        compiler_params=pltpu.CompilerParams(needs_layout_passes=False),
        scratch_types=[
            pltpu.VMEM((_RPW * _NPAD,), jnp.float32),
            pltpu.VMEM((32,), jnp.float32),
            pltpu.VMEM((16,), jnp.float32),
            pltpu.SemaphoreType.DMA,
        ],
    )
    def k(score_hbm, loc_hbm, out_hbm, score_v, d_v, out_v, sem):
        c = lax.axis_index("c")
        s = lax.axis_index("s")
        w = s * 2 + c
        base = w * _RPW
        pltpu.sync_copy(
            score_hbm.at[pl.ds(base * _NPAD, _RPW * _NPAD)], score_v)

        lane = lax.iota(jnp.int32, 16)
        row_l = lane >> 1
        quad_l = lane >> 2
        klane = lane & 3

        def _allreduce(v, binop):
            for step in (1, 2, 4, 8):
                shuf = v.at[lane ^ step].get(mode="promise_in_bounds")
                v = binop(v, shuf)
            return v

        def _pick(v, pos):
            return v.at[pos].get(mode="promise_in_bounds")

        def _scan_rows(rows):
            idx_quad = jnp.zeros((16,), jnp.int32)
            for j, r in enumerate(rows):
                vmax = score_v[pl.ds(r * _NPAD, 16)]
                vidx = lane
                for chunk in range(1, _NCHUNK):
                    v = score_v[pl.ds(r * _NPAD + chunk * 16, 16)]
                    gt = v > vmax
                    vmax = jnp.maximum(vmax, v)
                    vidx = jnp.where(gt, lane + chunk * 16, vidx)
                m = _allreduce(vmax, jnp.maximum)
                cand = jnp.where(vmax == m, vidx, jnp.int32(2**30))
                idx_vec = _allreduce(cand, jnp.minimum)
                idx_quad = jnp.where(quad_l == j, idx_vec, idx_quad)
            return idx_quad

        iq0 = _scan_rows(range(0, 4))
        iv0 = (base + quad_l) * (4 * _N) + klane * _N + iq0
        cp0 = pltpu.async_copy(loc_hbm.at[iv0], d_v.at[pl.ds(0, 16)], sem)
        iq1 = _scan_rows(range(4, 8))
        iv1 = (base + 4 + quad_l) * (4 * _N) + klane * _N + iq1
        cp1 = pltpu.async_copy(loc_hbm.at[iv1], d_v.at[pl.ds(16, 16)], sem)

        kbit = lane & 1
        pairpos = (((row_l & 3) << 2) | kbit)
        idx_pair = jnp.where(lane < 8, _pick(iq0, pairpos), _pick(iq1, pairpos))
        sel = jnp.where(kbit == 0, idx_pair % _SIZE, idx_pair // _SIZE)
        p = sel.astype(jnp.float32) * jnp.float32(_STRIDE) + jnp.float32(_ORI)

        cp0.wait()
        cp1.wait()
        g1 = plsc.load_gather(d_v, [(row_l << 2) | kbit])
        g2 = plsc.load_gather(d_v, [((row_l << 2) | kbit) + 2])
        out_v[...] = p + (g2 - g1) * jnp.float32(0.5)
        pltpu.sync_copy(out_v, out_hbm.at[pl.ds(base * 2, 16)])

    return k(score, locr)


def kernel(cls, loc):
    score = cls.reshape(_B, 2, _N)[:, 1, :]
    score = jnp.pad(score, ((0, 0), (0, _NPAD - _N)),
                    constant_values=float("-inf"))
    out = _polar_pick_sc(score.reshape(-1), loc.reshape(-1))
    return out.reshape(_B, 2)
